# baseline (speedup 1.0000x reference)
import functools

import jax
import jax.numpy as jnp
from jax import lax
from jax.experimental import pallas as pl
from jax.experimental.pallas import tpu as pltpu
from jax.experimental.pallas import tpu_sc as plsc

VOCAB = 1000
EMBED = 64
BATCH = 16384

NC = 2
NS = 16
NW = NC * NS
B_PER_W = BATCH // NW
CHUNK = 128
NCHUNK = B_PER_W // CHUNK
LANES = 16

_MESH = plsc.VectorSubcoreMesh(core_axis_name="c", subcore_axis_name="s")


@functools.partial(
    pl.kernel,
    mesh=_MESH,
    out_type=jax.ShapeDtypeStruct((NW, NCHUNK, CHUNK, EMBED), jnp.float32),
    scratch_types=[
        pltpu.VMEM((NCHUNK, CHUNK), jnp.int32),
        pltpu.VMEM((NCHUNK, CHUNK), jnp.float32),
        pltpu.VMEM((NCHUNK, CHUNK, EMBED), jnp.float32),
        pltpu.SemaphoreType.DMA,
        pltpu.SemaphoreType.DMA,
    ],
    compiler_params=pltpu.CompilerParams(
        use_tc_tiling_on_sc=False, needs_layout_passes=False
    ),
)
def _embed_lookup(idx_hbm, table_hbm, out_hbm, idx_v, mask_v, rows_v, gsem, osem):
    wid = lax.axis_index("s") * NC + lax.axis_index("c")
    pltpu.sync_copy(idx_hbm.at[pl.ds(wid * NCHUNK, NCHUNK)], idx_v)
    gathers = []
    counts = []
    for j in range(NCHUNK):
        cnt = jnp.int32(0)
        for i in range(CHUNK // LANES):
            v = idx_v[j, pl.ds(i * LANES, LANES)]
            z = v == 0
            mask_v[j, pl.ds(i * LANES, LANES)] = jnp.where(z, 0.0, 1.0)
            cnt = cnt + jnp.sum(jnp.where(z, 1, 0))
        counts.append(cnt)
        gathers.append(pltpu.async_copy(table_hbm.at[idx_v.at[j]], rows_v.at[j], gsem))
    writes = []
    for j in range(NCHUNK):
        gathers[j].wait()

        @pl.when(counts[j] > 0)
        def _fixup(j=j):
            def body(r, carry):
                m = plsc.load_gather(
                    mask_v.at[j], [jnp.full((LANES,), r, jnp.int32)]
                )
                for d in range(EMBED // LANES):
                    rows_v[j, r, pl.ds(d * LANES, LANES)] = (
                        rows_v[j, r, pl.ds(d * LANES, LANES)] * m
                    )
                return carry

            lax.fori_loop(0, CHUNK, body, jnp.int32(0))

        writes.append(pltpu.async_copy(rows_v.at[j], out_hbm.at[wid, j], osem))
    for c in writes:
        c.wait()


def kernel(age, table):
    idx = age.astype(jnp.int32).reshape(NW * NCHUNK, CHUNK)
    out = _embed_lookup(idx, table)
    return out.reshape(BATCH, EMBED)


# speedup vs baseline: 1.0043x; 1.0043x over previous
import functools

import jax
import jax.numpy as jnp
from jax import lax
from jax.experimental import pallas as pl
from jax.experimental.pallas import tpu as pltpu
from jax.experimental.pallas import tpu_sc as plsc

VOCAB = 1000
EMBED = 64
BATCH = 16384

NC = 2
NS = 16
NW = NC * NS
B_PER_W = BATCH // NW
CHUNK = 128
NCHUNK = B_PER_W // CHUNK
LANES = 16

_MESH = plsc.VectorSubcoreMesh(core_axis_name="c", subcore_axis_name="s")


@functools.partial(
    pl.kernel,
    mesh=_MESH,
    out_type=jax.ShapeDtypeStruct((BATCH, EMBED), jnp.float32),
    scratch_types=[
        pltpu.VMEM((B_PER_W,), jnp.int32),
        pltpu.VMEM((B_PER_W,), jnp.float32),
        pltpu.VMEM((B_PER_W, EMBED), jnp.float32),
        pltpu.SemaphoreType.DMA,
        pltpu.SemaphoreType.DMA,
    ],
    compiler_params=pltpu.CompilerParams(
        use_tc_tiling_on_sc=False, needs_layout_passes=False
    ),
)
def _embed_lookup(idx_hbm, table_hbm, out_hbm, idx_v, mask_v, rows_v, gsem, osem):
    wid = lax.axis_index("s") * NC + lax.axis_index("c")
    base = wid * B_PER_W
    pltpu.sync_copy(idx_hbm.at[pl.ds(base, B_PER_W)], idx_v)
    gathers = []
    counts = []
    for j in range(NCHUNK):
        cnt = jnp.int32(0)
        for i in range(CHUNK // LANES):
            v = idx_v[pl.ds(j * CHUNK + i * LANES, LANES)]
            z = v == 0
            mask_v[pl.ds(j * CHUNK + i * LANES, LANES)] = jnp.where(z, 0.0, 1.0)
            cnt = cnt + jnp.sum(jnp.where(z, 1, 0))
        counts.append(cnt)
        gathers.append(
            pltpu.async_copy(
                table_hbm.at[idx_v.at[pl.ds(j * CHUNK, CHUNK)]],
                rows_v.at[pl.ds(j * CHUNK, CHUNK)],
                gsem,
            )
        )
    writes = []
    for j in range(NCHUNK):
        gathers[j].wait()

        @pl.when(counts[j] > 0)
        def _fixup(j=j):
            def body(r, carry):
                m = plsc.load_gather(mask_v, [jnp.full((LANES,), r, jnp.int32)])
                for d in range(EMBED // LANES):
                    rows_v[r, pl.ds(d * LANES, LANES)] = (
                        rows_v[r, pl.ds(d * LANES, LANES)] * m
                    )
                return carry

            lax.fori_loop(j * CHUNK, (j + 1) * CHUNK, body, jnp.int32(0))

        writes.append(
            pltpu.async_copy(
                rows_v.at[pl.ds(j * CHUNK, CHUNK)],
                out_hbm.at[pl.ds(base + j * CHUNK, CHUNK)],
                osem,
            )
        )
    for c in writes:
        c.wait()


def kernel(age, table):
    return _embed_lookup(age.astype(jnp.int32), table)


# speedup vs baseline: 1.2484x; 1.2430x over previous
import functools

import jax
import jax.numpy as jnp
from jax import lax
from jax.experimental import pallas as pl
from jax.experimental.pallas import tpu as pltpu
from jax.experimental.pallas import tpu_sc as plsc

VOCAB = 1000
EMBED = 64
BATCH = 16384

NC = 2
NS = 16
NW = NC * NS
B_PER_W = BATCH // NW
CHUNK = 128
NCHUNK = B_PER_W // CHUNK
LANES = 16

_MESH = plsc.VectorSubcoreMesh(core_axis_name="c", subcore_axis_name="s")


@functools.partial(
    pl.kernel,
    mesh=_MESH,
    out_type=jax.ShapeDtypeStruct((BATCH, 2 * EMBED), jnp.float32),
    scratch_types=[
        pltpu.VMEM((B_PER_W,), jnp.int32),
        pltpu.VMEM((B_PER_W,), jnp.float32),
        pltpu.VMEM((B_PER_W, EMBED), jnp.float32),
        pltpu.SemaphoreType.DMA,
        pltpu.SemaphoreType.DMA,
    ],
    compiler_params=pltpu.CompilerParams(
        use_tc_tiling_on_sc=False, needs_layout_passes=False
    ),
)
def _embed_lookup(idx_hbm, table_hbm, out_hbm, idx_v, mask_v, rows_v, gsem, osem):
    wid = lax.axis_index("s") * NC + lax.axis_index("c")
    base = wid * B_PER_W
    pltpu.sync_copy(idx_hbm.at[pl.ds(base, B_PER_W)], idx_v)
    gathers = []
    counts = []
    for j in range(NCHUNK):
        cnt = jnp.int32(0)
        for i in range(CHUNK // LANES):
            v = idx_v[pl.ds(j * CHUNK + i * LANES, LANES)]
            z = v == 0
            mask_v[pl.ds(j * CHUNK + i * LANES, LANES)] = jnp.where(z, 0.0, 1.0)
            cnt = cnt + jnp.sum(jnp.where(z, 1, 0))
        counts.append(cnt)
        gathers.append(
            pltpu.async_copy(
                table_hbm.at[idx_v.at[pl.ds(j * CHUNK, CHUNK)]],
                rows_v.at[pl.ds(j * CHUNK, CHUNK)],
                gsem,
            )
        )
    writes = []
    for j in range(NCHUNK):
        gathers[j].wait()

        @pl.when(counts[j] > 0)
        def _fixup(j=j):
            def body(r, carry):
                m = plsc.load_gather(mask_v, [jnp.full((LANES,), r, jnp.int32)])
                for d in range(EMBED // LANES):
                    rows_v[r, pl.ds(d * LANES, LANES)] = (
                        rows_v[r, pl.ds(d * LANES, LANES)] * m
                    )
                return carry

            lax.fori_loop(j * CHUNK, (j + 1) * CHUNK, body, jnp.int32(0))

        writes.append(
            pltpu.async_copy(
                rows_v.at[pl.ds(j * CHUNK, CHUNK)],
                out_hbm.at[pl.ds(base + j * CHUNK, CHUNK), pl.ds(0, EMBED)],
                osem,
            )
        )
    for c in writes:
        c.wait()


def kernel(age, table):
    out = _embed_lookup(age.astype(jnp.int32), table)
    return out[:, :EMBED]
